# baseline (speedup 1.0000x reference)
import functools

import jax
import jax.numpy as jnp
import numpy as np
from jax import lax
from jax.experimental import pallas as pl
from jax.experimental.pallas import tpu as pltpu
from jax.experimental.pallas import tpu_sc as plsc

VOCAB = 1000000
DIM = 64
BATCH = 16384
WINDOW = 20
NEG = 5
BW = BATCH * WINDOW
PAD = 8

NW = 32
PAIR_PER_SUB = BW // NW
CHUNK_PAIRS = 64
CHUNK_NOISE = CHUNK_PAIRS * NEG
BLK_CHUNKS = 20
NSLOT = 3
BLK_PAIRS = CHUNK_PAIRS * BLK_CHUNKS
BLK_WIN = BLK_PAIRS // WINDOW
NBLK = PAIR_PER_SUB // BLK_PAIRS
GROUPS = CHUNK_PAIRS // 16
DOTS_CHUNK = CHUNK_PAIRS * PAD


def _sc_dots(node_emb, ctx_emb, inp_idx, out_idx, noise_idx, winrow):
    mesh = plsc.VectorSubcoreMesh(core_axis_name="c", subcore_axis_name="s")

    @functools.partial(
        pl.kernel,
        out_type=jax.ShapeDtypeStruct((BW * PAD,), jnp.float32),
        mesh=mesh,
        compiler_params=pltpu.CompilerParams(needs_layout_passes=False,
                                             use_tc_tiling_on_sc=False),
        scratch_types=[
            pltpu.VMEM((BLK_WIN,), jnp.int32),
            pltpu.VMEM((BLK_PAIRS,), jnp.int32),
            pltpu.VMEM((NEG, BLK_PAIRS), jnp.int32),

            pltpu.VMEM((BLK_PAIRS,), jnp.int32),
            pltpu.VMEM((BLK_WIN, DIM), jnp.float32),
            pltpu.VMEM((CHUNK_PAIRS, DIM), jnp.float32),
            pltpu.VMEM((CHUNK_PAIRS, DIM), jnp.float32),
            pltpu.VMEM((CHUNK_PAIRS, DIM), jnp.float32),
            pltpu.VMEM((CHUNK_NOISE, DIM), jnp.float32),
            pltpu.VMEM((CHUNK_NOISE, DIM), jnp.float32),
            pltpu.VMEM((CHUNK_NOISE, DIM), jnp.float32),
            pltpu.VMEM((DOTS_CHUNK,), jnp.float32),
            pltpu.VMEM((DOTS_CHUNK,), jnp.float32),
            pltpu.VMEM((DOTS_CHUNK,), jnp.float32),
            pltpu.SemaphoreType.DMA,
            pltpu.SemaphoreType.DMA,
            pltpu.SemaphoreType.DMA,
            pltpu.SemaphoreType.DMA,
        ],
    )
    def k(node_hbm, ctx_hbm, inp_hbm, out_hbm, noise_hbm, winrow_hbm,
          dots_hbm, inp_idx_v, out_idx_v, noise_idx_v, winrow_v,
          inp_rows, out_rows0, out_rows1, out_rows2, noise_rows0,
          noise_rows1, noise_rows2, dots0, dots1, dots2, sem_g0, sem_g1,
          sem_g2, sem_wb):
        wid = lax.axis_index("s") * 2 + lax.axis_index("c")
        pltpu.sync_copy(winrow_hbm, winrow_v)
        lam = lax.iota(jnp.int32, 16)
        out_rows = (out_rows0, out_rows1, out_rows2)
        noise_rows = (noise_rows0, noise_rows1, noise_rows2)
        dots_v = (dots0, dots1, dots2)
        sem_g = (sem_g0, sem_g1, sem_g2)

        def issue_gathers(cc, slot):
            hs = [pltpu.async_copy(
                ctx_hbm.at[out_idx_v.at[pl.ds(cc * CHUNK_PAIRS,
                                              CHUNK_PAIRS)]],
                out_rows[slot], sem_g[slot])]
            for n in range(NEG):
                hs.append(pltpu.async_copy(
                    ctx_hbm.at[noise_idx_v.at[n].at[pl.ds(cc * CHUNK_PAIRS,
                                                          CHUNK_PAIRS)]],
                    noise_rows[slot].at[pl.ds(n * CHUNK_PAIRS, CHUNK_PAIRS)],
                    sem_g[slot]))
            return hs

        DUNROLL = 8

        def compute_chunk(cc, slot):
            def group_body(g, carry):
                p16 = cc * CHUNK_PAIRS + g * 16
                pair16 = g * 16 + lam
                win_v = winrow_v[pl.ds(p16, 16)]
                pair_n = [n * CHUNK_PAIRS + pair16 for n in range(NEG)]

                def d_body(i, accs):
                    accs = list(accs)
                    for u in range(DUNROLL):
                        dv = (lam + i * DUNROLL + u) & (DIM - 1)
                        a = plsc.load_gather(inp_rows, [win_v, dv])
                        o = plsc.load_gather(out_rows[slot], [pair16, dv])
                        accs[0] = accs[0] + a * o
                        for n in range(NEG):
                            x = plsc.load_gather(noise_rows[slot],
                                                 [pair_n[n], dv])
                            accs[1 + n] = accs[1 + n] - a * x
                    return tuple(accs)

                zero = jnp.zeros((16,), jnp.float32)
                accs = lax.fori_loop(0, DIM // DUNROLL, d_body, (zero,) * 6)
                base8 = pair16 * PAD
                for t in range(6):
                    plsc.store_scatter(dots_v[slot], [base8 + t], accs[t])
                return carry

            lax.fori_loop(0, GROUPS, group_body, 0)

        def block_body(b, carry):
            g_b = wid * NBLK + b
            wbase = g_b * BLK_WIN
            pbase = g_b * BLK_PAIRS
            pltpu.sync_copy(inp_hbm.at[pl.ds(wbase, BLK_WIN)], inp_idx_v)
            pltpu.sync_copy(out_hbm.at[pl.ds(pbase, BLK_PAIRS)], out_idx_v)
            pltpu.sync_copy(noise_hbm.at[:, pl.ds(pbase, BLK_PAIRS)],
                            noise_idx_v)
            h_inp = pltpu.async_copy(node_hbm.at[inp_idx_v], inp_rows,
                                     sem_g0)
            pending = {0: [h_inp] + issue_gathers(0, 0),
                       1: issue_gathers(1, 1), 2: []}
            wb = {0: None, 1: None, 2: None}
            for cc in range(BLK_CHUNKS):
                slot = cc % NSLOT
                if cc + 2 < BLK_CHUNKS:
                    nslot = (cc + 2) % NSLOT
                    pending[nslot] = issue_gathers(cc + 2, nslot)
                for h in pending[slot]:
                    h.wait()
                if wb[slot] is not None:
                    wb[slot].wait()
                compute_chunk(cc, slot)
                wb[slot] = pltpu.async_copy(
                    dots_v[slot],
                    dots_hbm.at[pl.ds((g_b * BLK_CHUNKS + cc) * DOTS_CHUNK,
                                      DOTS_CHUNK)],
                    sem_wb)
            for s in range(NSLOT):
                if wb[s] is not None:
                    wb[s].wait()
            return carry

        lax.fori_loop(0, NBLK, block_body, 0)

    return k(node_emb, ctx_emb, inp_idx, out_idx, noise_idx, winrow)


_TC_ROWS = 2560
_TC_COLS = 1024
_TC_BLK = 256
_TC_GRID = _TC_ROWS // _TC_BLK


def _tc_reduce_body(x_ref, o_ref):
    i = pl.program_id(0)
    x = x_ref[...]
    z = jnp.minimum(x, 0.0) - jnp.log1p(jnp.exp(-jnp.abs(x)))
    col = lax.broadcasted_iota(jnp.int32, (_TC_BLK, _TC_COLS), 1)
    z = jnp.where((col % PAD) < 6, z, 0.0)
    s = jnp.sum(z)

    @pl.when(i == 0)
    def _():
        o_ref[0, 0] = 0.0

    o_ref[0, 0] += s


def _tc_reduce(dots):
    dots2d = jnp.reshape(dots, (_TC_ROWS, _TC_COLS))
    return pl.pallas_call(
        _tc_reduce_body,
        grid=(_TC_GRID,),
        in_specs=[pl.BlockSpec((_TC_BLK, _TC_COLS), lambda i: (i, 0))],
        out_specs=pl.BlockSpec(memory_space=pltpu.SMEM),
        out_shape=jax.ShapeDtypeStruct((1, 1), jnp.float32),
    )(dots2d)


def _pad2(table):
    wide = jnp.concatenate([table, jnp.zeros((VOCAB, DIM), table.dtype)],
                           axis=1)
    return jnp.reshape(wide, (2 * VOCAB, DIM))


def kernel(input_labels, out_labels, noise_indices, node_emb, ctx_emb):
    node_emb = _pad2(node_emb)
    ctx_emb = _pad2(ctx_emb)
    inp_idx = jnp.left_shift(input_labels.astype(jnp.int32), 1)
    out_idx = jnp.left_shift(
        jnp.reshape(out_labels, (BW,)).astype(jnp.int32), 1)
    noise_idx = jnp.left_shift(jnp.transpose(noise_indices).astype(jnp.int32),
                               1)
    winrow = jnp.asarray(np.arange(BLK_PAIRS) // WINDOW, dtype=jnp.int32)
    dots = _sc_dots(node_emb, ctx_emb, inp_idx, out_idx, noise_idx, winrow)
    total = _tc_reduce(dots)
    return -total[0, 0] / BATCH


# speedup vs baseline: 1.0150x; 1.0150x over previous
import functools

import jax
import jax.numpy as jnp
import numpy as np
from jax import lax
from jax.experimental import pallas as pl
from jax.experimental.pallas import tpu as pltpu
from jax.experimental.pallas import tpu_sc as plsc

VOCAB = 1000000
DIM = 64
BATCH = 16384
WINDOW = 20
NEG = 5
BW = BATCH * WINDOW
PAD = 8

NW = 32
PAIR_PER_SUB = BW // NW
CHUNK_PAIRS = 128
CHUNK_NOISE = CHUNK_PAIRS * NEG
BLK_CHUNKS = 10
NSLOT = 2
BLK_PAIRS = CHUNK_PAIRS * BLK_CHUNKS
BLK_WIN = BLK_PAIRS // WINDOW
NBLK = PAIR_PER_SUB // BLK_PAIRS
GROUPS = CHUNK_PAIRS // 16
DOTS_CHUNK = CHUNK_PAIRS * PAD


def _sc_dots(node_emb, ctx_emb, inp_idx, out_idx, noise_idx, winrow):
    mesh = plsc.VectorSubcoreMesh(core_axis_name="c", subcore_axis_name="s")

    @functools.partial(
        pl.kernel,
        out_type=jax.ShapeDtypeStruct((BW * PAD,), jnp.float32),
        mesh=mesh,
        compiler_params=pltpu.CompilerParams(needs_layout_passes=False,
                                             use_tc_tiling_on_sc=False),
        scratch_types=[
            pltpu.VMEM((BLK_WIN,), jnp.int32),
            pltpu.VMEM((BLK_PAIRS,), jnp.int32),
            pltpu.VMEM((NEG, BLK_PAIRS), jnp.int32),

            pltpu.VMEM((BLK_PAIRS,), jnp.int32),
            pltpu.VMEM((BLK_WIN, DIM), jnp.float32),
            pltpu.VMEM((CHUNK_PAIRS, DIM), jnp.float32),
            pltpu.VMEM((CHUNK_PAIRS, DIM), jnp.float32),
            pltpu.VMEM((CHUNK_NOISE, DIM), jnp.float32),
            pltpu.VMEM((CHUNK_NOISE, DIM), jnp.float32),
            pltpu.VMEM((DOTS_CHUNK,), jnp.float32),
            pltpu.VMEM((DOTS_CHUNK,), jnp.float32),
            pltpu.SemaphoreType.DMA,
            pltpu.SemaphoreType.DMA,
            pltpu.SemaphoreType.DMA,
        ],
    )
    def k(node_hbm, ctx_hbm, inp_hbm, out_hbm, noise_hbm, winrow_hbm,
          dots_hbm, inp_idx_v, out_idx_v, noise_idx_v, winrow_v,
          inp_rows, out_rows0, out_rows1, noise_rows0,
          noise_rows1, dots0, dots1, sem_g0, sem_g1, sem_wb):
        wid = lax.axis_index("s") * 2 + lax.axis_index("c")
        pltpu.sync_copy(winrow_hbm, winrow_v)
        lam = lax.iota(jnp.int32, 16)
        out_rows = (out_rows0, out_rows1)
        noise_rows = (noise_rows0, noise_rows1)
        dots_v = (dots0, dots1)
        sem_g = (sem_g0, sem_g1)

        def issue_gathers(cc, slot):
            hs = [pltpu.async_copy(
                ctx_hbm.at[out_idx_v.at[pl.ds(cc * CHUNK_PAIRS,
                                              CHUNK_PAIRS)]],
                out_rows[slot], sem_g[slot])]
            for n in range(NEG):
                hs.append(pltpu.async_copy(
                    ctx_hbm.at[noise_idx_v.at[n].at[pl.ds(cc * CHUNK_PAIRS,
                                                          CHUNK_PAIRS)]],
                    noise_rows[slot].at[pl.ds(n * CHUNK_PAIRS, CHUNK_PAIRS)],
                    sem_g[slot]))
            return hs

        DUNROLL = 8

        def compute_chunk(cc, slot):
            def group_body(g, carry):
                p16 = cc * CHUNK_PAIRS + g * 16
                pair16 = g * 16 + lam
                win_v = winrow_v[pl.ds(p16, 16)]
                pair_n = [n * CHUNK_PAIRS + pair16 for n in range(NEG)]

                def d_body(i, accs):
                    accs = list(accs)
                    for u in range(DUNROLL):
                        dv = (lam + i * DUNROLL + u) & (DIM - 1)
                        a = plsc.load_gather(inp_rows, [win_v, dv])
                        o = plsc.load_gather(out_rows[slot], [pair16, dv])
                        accs[0] = accs[0] + a * o
                        for n in range(NEG):
                            x = plsc.load_gather(noise_rows[slot],
                                                 [pair_n[n], dv])
                            accs[1 + n] = accs[1 + n] - a * x
                    return tuple(accs)

                zero = jnp.zeros((16,), jnp.float32)
                accs = lax.fori_loop(0, DIM // DUNROLL, d_body, (zero,) * 6)
                base8 = pair16 * PAD
                for t in range(6):
                    plsc.store_scatter(dots_v[slot], [base8 + t], accs[t])
                return carry

            lax.fori_loop(0, GROUPS, group_body, 0)

        def block_body(b, carry):
            g_b = wid * NBLK + b
            wbase = g_b * BLK_WIN
            pbase = g_b * BLK_PAIRS
            pltpu.sync_copy(inp_hbm.at[pl.ds(wbase, BLK_WIN)], inp_idx_v)
            pltpu.sync_copy(out_hbm.at[pl.ds(pbase, BLK_PAIRS)], out_idx_v)
            pltpu.sync_copy(noise_hbm.at[:, pl.ds(pbase, BLK_PAIRS)],
                            noise_idx_v)
            h_inp = pltpu.async_copy(node_hbm.at[inp_idx_v], inp_rows,
                                     sem_g0)
            pending = {0: [h_inp] + issue_gathers(0, 0), 1: []}
            wb = {0: None, 1: None}
            for cc in range(BLK_CHUNKS):
                slot = cc % NSLOT
                if cc + 1 < BLK_CHUNKS:
                    pending[1 - slot] = issue_gathers(cc + 1, 1 - slot)
                for h in pending[slot]:
                    h.wait()
                if wb[slot] is not None:
                    wb[slot].wait()
                compute_chunk(cc, slot)
                wb[slot] = pltpu.async_copy(
                    dots_v[slot],
                    dots_hbm.at[pl.ds((g_b * BLK_CHUNKS + cc) * DOTS_CHUNK,
                                      DOTS_CHUNK)],
                    sem_wb)
            for s in range(NSLOT):
                if wb[s] is not None:
                    wb[s].wait()
            return carry

        lax.fori_loop(0, NBLK, block_body, 0)

    return k(node_emb, ctx_emb, inp_idx, out_idx, noise_idx, winrow)


_TC_ROWS = 2560
_TC_COLS = 1024
_TC_BLK = 256
_TC_GRID = _TC_ROWS // _TC_BLK


def _tc_reduce_body(x_ref, o_ref):
    i = pl.program_id(0)
    x = x_ref[...]
    z = jnp.minimum(x, 0.0) - jnp.log1p(jnp.exp(-jnp.abs(x)))
    col = lax.broadcasted_iota(jnp.int32, (_TC_BLK, _TC_COLS), 1)
    z = jnp.where((col % PAD) < 6, z, 0.0)
    s = jnp.sum(z)

    @pl.when(i == 0)
    def _():
        o_ref[0, 0] = 0.0

    o_ref[0, 0] += s


def _tc_reduce(dots):
    dots2d = jnp.reshape(dots, (_TC_ROWS, _TC_COLS))
    return pl.pallas_call(
        _tc_reduce_body,
        grid=(_TC_GRID,),
        in_specs=[pl.BlockSpec((_TC_BLK, _TC_COLS), lambda i: (i, 0))],
        out_specs=pl.BlockSpec(memory_space=pltpu.SMEM),
        out_shape=jax.ShapeDtypeStruct((1, 1), jnp.float32),
    )(dots2d)


def _pad2(table):
    wide = jnp.concatenate([table, jnp.zeros((VOCAB, DIM), table.dtype)],
                           axis=1)
    return jnp.reshape(wide, (2 * VOCAB, DIM))


def kernel(input_labels, out_labels, noise_indices, node_emb, ctx_emb):
    node_emb = _pad2(node_emb)
    ctx_emb = _pad2(ctx_emb)
    inp_idx = jnp.left_shift(input_labels.astype(jnp.int32), 1)
    out_idx = jnp.left_shift(
        jnp.reshape(out_labels, (BW,)).astype(jnp.int32), 1)
    noise_idx = jnp.left_shift(jnp.transpose(noise_indices).astype(jnp.int32),
                               1)
    winrow = jnp.asarray(np.arange(BLK_PAIRS) // WINDOW, dtype=jnp.int32)
    dots = _sc_dots(node_emb, ctx_emb, inp_idx, out_idx, noise_idx, winrow)
    total = _tc_reduce(dots)
    return -total[0, 0] / BATCH
